# baseline (speedup 1.0000x reference)
import functools

import jax
import jax.numpy as jnp
from jax.experimental import pallas as pl
from jax.experimental.pallas import tpu as pltpu
from jax.experimental.pallas import tpu_sc as plsc

N = 10000
D = 128
E = 320000
K_HOPS = 2
NC = 2
NS = 16
NW = NC * NS
EPW_RAW = E // NW
C = 128
NCHUNK = -(-EPW_RAW // C)
EPW = NCHUNK * C
N_PAD = 10016
DUMP = N
ROWS_PER_SUB = 1000
DEG_W = 128
DEG_R = N_PAD * DEG_W // 128


def _vector_mesh():
    return plsc.VectorSubcoreMesh(core_axis_name="c", subcore_axis_name="s")


def _deg_sc(dstp, zeros_nd, ones_cw):
    @functools.partial(
        pl.kernel,
        out_type=jax.ShapeDtypeStruct((NC, N, DEG_W), jnp.float32),
        mesh=_vector_mesh(),
        scratch_types=[
            pltpu.VMEM((2, C), jnp.int32),
            pltpu.VMEM((C, DEG_W), jnp.float32),
            pltpu.VMEM_SHARED((N_PAD, DEG_W), jnp.float32),
            pltpu.SemaphoreType.DMA,
            pltpu.SemaphoreType.DMA,
        ],
    )
    def k(dst_hbm, z_hbm, ones_hbm, out_hbm, didx, ones_v, acc, sem0, sem1):
        cid = jax.lax.axis_index("c")
        sid = jax.lax.axis_index("s")
        base0 = (cid * NS + sid) * EPW
        sems = (sem0, sem1)

        didx_d = {}

        def issue_idx(c):
            s = c & 1
            didx_d[c] = pltpu.async_copy(
                dst_hbm.at[pl.ds(base0 + c * C, C)], didx.at[s], sems[s]
            )

        issue_idx(0)
        if NCHUNK > 1:
            issue_idx(1)
        pltpu.sync_copy(ones_hbm, ones_v)

        @pl.when(sid < 10)
        def _():
            r = sid * ROWS_PER_SUB
            pltpu.sync_copy(
                z_hbm.at[pl.ds(r, ROWS_PER_SUB)], acc.at[pl.ds(r, ROWS_PER_SUB)]
            )

        @pl.when(sid == 10)
        def _():
            pltpu.sync_copy(
                z_hbm.at[pl.ds(N, N_PAD - N)], acc.at[pl.ds(N, N_PAD - N)]
            )

        plsc.subcore_barrier()

        for c in range(NCHUNK):
            s = c & 1
            didx_d[c].wait()
            pltpu.sync_copy(ones_v, acc.at[didx.at[s]], add=True)
            if c + 2 < NCHUNK:
                issue_idx(c + 2)

        plsc.subcore_barrier()

        @pl.when(sid < 10)
        def _():
            r = sid * ROWS_PER_SUB
            pltpu.sync_copy(
                acc.at[pl.ds(r, ROWS_PER_SUB)],
                out_hbm.at[cid, pl.ds(r, ROWS_PER_SUB)],
            )

    return k(dstp, zeros_nd, ones_cw)


def _hop_sc(h, srcp, dstp, zeros_nd):
    @functools.partial(
        pl.kernel,
        out_type=jax.ShapeDtypeStruct((NC, N, D), jnp.float32),
        mesh=_vector_mesh(),
        scratch_types=[
            pltpu.VMEM((2, C), jnp.int32),
            pltpu.VMEM((2, C), jnp.int32),
            pltpu.VMEM((2, C, D), jnp.float32),
            pltpu.VMEM_SHARED((N_PAD, D), jnp.float32),
            pltpu.SemaphoreType.DMA,
            pltpu.SemaphoreType.DMA,
            pltpu.SemaphoreType.DMA,
            pltpu.SemaphoreType.DMA,
        ],
    )
    def k(h_hbm, src_hbm, dst_hbm, z_hbm, out_hbm,
          sidx, didx, rows, acc, semi0, semi1, semg0, semg1):
        cid = jax.lax.axis_index("c")
        sid = jax.lax.axis_index("s")
        base0 = (cid * NS + sid) * EPW
        semi = (semi0, semi1)
        semg = (semg0, semg1)

        sidx_d, didx_d, g_d = {}, {}, {}

        def issue_idx(c):
            s = c & 1
            b = base0 + c * C
            sidx_d[c] = pltpu.async_copy(
                src_hbm.at[pl.ds(b, C)], sidx.at[s], semi[s]
            )
            didx_d[c] = pltpu.async_copy(
                dst_hbm.at[pl.ds(b, C)], didx.at[s], semi[s]
            )

        def issue_gather(c):
            s = c & 1
            g_d[c] = pltpu.async_copy(h_hbm.at[sidx.at[s]], rows.at[s], semg[s])

        issue_idx(0)
        if NCHUNK > 1:
            issue_idx(1)

        @pl.when(sid < 10)
        def _():
            r = sid * ROWS_PER_SUB
            pltpu.sync_copy(
                z_hbm.at[pl.ds(r, ROWS_PER_SUB)], acc.at[pl.ds(r, ROWS_PER_SUB)]
            )

        @pl.when(sid == 10)
        def _():
            pltpu.sync_copy(
                z_hbm.at[pl.ds(N, N_PAD - N)], acc.at[pl.ds(N, N_PAD - N)]
            )

        sidx_d[0].wait()
        didx_d[0].wait()
        issue_gather(0)
        plsc.subcore_barrier()

        for c in range(NCHUNK):
            s = c & 1
            g_d[c].wait()
            if c + 1 < NCHUNK:
                sidx_d[c + 1].wait()
                didx_d[c + 1].wait()
                issue_gather(c + 1)
            pltpu.sync_copy(rows.at[s], acc.at[didx.at[s]], add=True)
            if c + 2 < NCHUNK:
                issue_idx(c + 2)

        plsc.subcore_barrier()

        @pl.when(sid < 10)
        def _():
            r = sid * ROWS_PER_SUB
            pltpu.sync_copy(
                acc.at[pl.ds(r, ROWS_PER_SUB)],
                out_hbm.at[cid, pl.ds(r, ROWS_PER_SUB)],
            )

    return k(h, srcp, dstp, zeros_nd)


def _prep_tc(node_feat, degp):
    def body(nf_ref, degp_ref, hs_ref, dn_ref):
        deg = degp_ref[0, :, 0:1] + degp_ref[1, :, 0:1]
        dn = jnp.where(deg > 0, jax.lax.rsqrt(jnp.maximum(deg, 1.0)), 0.0)
        dn_ref[...] = dn
        hs_ref[...] = nf_ref[...] * dn

    return pl.pallas_call(
        body,
        out_shape=(
            jax.ShapeDtypeStruct((N, D), jnp.float32),
            jax.ShapeDtypeStruct((N, 1), jnp.float32),
        ),
    )(node_feat, degp)


def _standardize(t):
    mu = jnp.mean(t, axis=0, keepdims=True)
    c = t - mu
    sd = jnp.sqrt(jnp.sum(c * c, axis=0, keepdims=True) / (N - 1))
    return c / (sd + 1e-5)


def _mid_tc(p, dn):
    def body(p_ref, dn_ref, out_ref):
        t = (p_ref[0] + p_ref[1]) * dn_ref[...]
        out_ref[...] = _standardize(t) * dn_ref[...]

    return pl.pallas_call(
        body, out_shape=jax.ShapeDtypeStruct((N, D), jnp.float32)
    )(p, dn)


def _final_tc(p, dn, W, b2):
    def body(p_ref, dn_ref, w_ref, b_ref, out_ref):
        t = (p_ref[0] + p_ref[1]) * dn_ref[...]
        t = _standardize(t)
        out_ref[...] = (
            jnp.dot(t, w_ref[...], preferred_element_type=jnp.float32)
            + b_ref[...]
        )

    return pl.pallas_call(
        body, out_shape=jax.ShapeDtypeStruct((N, D), jnp.float32)
    )(p, dn, W, b2)


def kernel(node_feat, edge_index, W, b):
    src = edge_index[0]
    dst = edge_index[1]
    pad = EPW - EPW_RAW
    srcp = jnp.pad(src.reshape(NW, EPW_RAW), ((0, 0), (0, pad))).reshape(-1)
    dstp = jnp.pad(
        dst.reshape(NW, EPW_RAW), ((0, 0), (0, pad)), constant_values=DUMP
    ).reshape(-1)
    zeros_nd = jnp.zeros((N_PAD, D), jnp.float32)
    ones_cw = jnp.ones((C, DEG_W), jnp.float32)

    degp = _deg_sc(dstp, zeros_nd, ones_cw)
    hs, dn = _prep_tc(node_feat, degp)
    p = None
    for hop in range(K_HOPS):
        p = _hop_sc(hs, srcp, dstp, zeros_nd)
        if hop < K_HOPS - 1:
            hs = _mid_tc(p, dn)
    return _final_tc(p, dn, W, b.reshape(1, D))
